# baseline (speedup 1.0000x reference)
import functools

import jax
import jax.numpy as jnp
from jax import lax
from jax.experimental import pallas as pl
from jax.experimental.pallas import tpu as pltpu
from jax.experimental.pallas import tpu_sc as plsc

N = 10000
E = 160000
D = 256
R = 8
B = 8
H = 128

NSUB = 16
CHUNK = 128
CH = 80
NPH = 2
CHP = CH // NPH
EPAD = NSUB * CH * CHUNK
TRASH = N
ACC_ROWS = N + 8
MB = 1000
NMB = N // MB


def _prep_body(src_ref, et_ref, g_ref):
    g_ref[...] = et_ref[...] * N + src_ref[...]


def _prep_idx(src2, et2):
    return pl.pallas_call(
        _prep_body,
        out_shape=jax.ShapeDtypeStruct(src2.shape, jnp.int32),
    )(src2, et2)


def _mm_bias_body(x_ref, w_ref, b_ref, o_ref):
    xb = x_ref[...].astype(jnp.bfloat16)
    wb = w_ref[...].astype(jnp.bfloat16)
    o_ref[...] = jnp.dot(xb, wb, preferred_element_type=jnp.float32) + b_ref[...]


def _mm_bias(x, w, b2):
    return pl.pallas_call(
        _mm_bias_body,
        grid=(NMB,),
        in_specs=[
            pl.BlockSpec((MB, D), lambda m: (m, 0)),
            pl.BlockSpec((D, D), lambda m: (0, 0)),
            pl.BlockSpec((1, D), lambda m: (0, 0)),
        ],
        out_specs=pl.BlockSpec((MB, D), lambda m: (m, 0)),
        out_shape=jax.ShapeDtypeStruct((N, D), jnp.float32),
    )(x, w, b2)


def _mm_body(x_ref, w_ref, o_ref):
    xb = x_ref[...].astype(jnp.bfloat16)
    wb = w_ref[...].astype(jnp.bfloat16)
    o_ref[...] = jnp.dot(xb, wb, preferred_element_type=jnp.float32)


def _mm(x, w):
    return pl.pallas_call(
        _mm_body,
        grid=(NMB,),
        in_specs=[
            pl.BlockSpec((MB, D), lambda m: (m, 0)),
            pl.BlockSpec((D, D), lambda m: (0, 0)),
        ],
        out_specs=pl.BlockSpec((MB, D), lambda m: (m, 0)),
        out_shape=jax.ShapeDtypeStruct((N, D), jnp.float32),
    )(x, w)


def _basis_body(c_ref, bs_ref, o_ref):
    cb = c_ref[...].astype(jnp.bfloat16)
    bb = bs_ref[...].astype(jnp.bfloat16)
    o_ref[...] = jnp.dot(cb, bb, preferred_element_type=jnp.float32)


def _basis_w(coef, bases_flat):
    cols = 8192
    return pl.pallas_call(
        _basis_body,
        grid=(bases_flat.shape[1] // cols,),
        in_specs=[
            pl.BlockSpec((R, B), lambda i: (0, 0)),
            pl.BlockSpec((B, cols), lambda i: (0, i)),
        ],
        out_specs=pl.BlockSpec((R, cols), lambda i: (0, i)),
        out_shape=jax.ShapeDtypeStruct((R, bases_flat.shape[1]), jnp.float32),
    )(coef, bases_flat)


def _hrel_body(h_ref, w_ref, lo_ref, hi_ref):
    hb = h_ref[...].astype(jnp.bfloat16)
    wb = w_ref[0].astype(jnp.bfloat16)
    o = jnp.dot(hb, wb, preferred_element_type=jnp.float32)
    lo_ref[...] = o[:, :H]
    hi_ref[...] = o[:, H:]


def _hrel(h, w3):
    return pl.pallas_call(
        _hrel_body,
        grid=(NMB, R),
        in_specs=[
            pl.BlockSpec((MB, D), lambda m, r: (m, 0)),
            pl.BlockSpec((1, D, D), lambda m, r: (r, 0, 0)),
        ],
        out_specs=[
            pl.BlockSpec((MB, H), lambda m, r: (r * NMB + m, 0)),
            pl.BlockSpec((MB, H), lambda m, r: (r * NMB + m, 0)),
        ],
        out_shape=[
            jax.ShapeDtypeStruct((R * N, H), jnp.float32),
            jax.ShapeDtypeStruct((R * N, H), jnp.float32),
        ],
    )(h, w3)


def _ln_body(lo_ref, hi_ref, s_ref, bias_ref, g_ref, b_ref, o_ref, *, relu):
    o = jnp.concatenate([lo_ref[...], hi_ref[...]], axis=1) + s_ref[...] + bias_ref[...]
    mu = jnp.mean(o, axis=1, keepdims=True)
    d = o - mu
    var = jnp.mean(d * d, axis=1, keepdims=True)
    y = d * lax.rsqrt(var + 1e-5) * g_ref[...] + b_ref[...]
    if relu:
        y = jnp.maximum(y, 0.0)
    o_ref[...] = y


def _ln(agg_lo, agg_hi, s, bias2, g2, b2, relu):
    return pl.pallas_call(
        functools.partial(_ln_body, relu=relu),
        grid=(NMB,),
        in_specs=[
            pl.BlockSpec((MB, H), lambda m: (m, 0)),
            pl.BlockSpec((MB, H), lambda m: (m, 0)),
            pl.BlockSpec((MB, D), lambda m: (m, 0)),
            pl.BlockSpec((1, D), lambda m: (0, 0)),
            pl.BlockSpec((1, D), lambda m: (0, 0)),
            pl.BlockSpec((1, D), lambda m: (0, 0)),
        ],
        out_specs=pl.BlockSpec((MB, D), lambda m: (m, 0)),
        out_shape=jax.ShapeDtypeStruct((N, D), jnp.float32),
    )(agg_lo, agg_hi, s, bias2, g2, b2)


def _sc_agg_body(hlo_hbm, hhi_hbm, gidx_hbm, dst_hbm, zer_hbm,
                 outlo_hbm, outhi_hbm,
                 gidx_v, dst_v, rows_a, rows_b, acc, sem_a, sem_b):
    c = lax.axis_index("c")
    s = lax.axis_index("s")

    @pl.when(s < 10)
    def _():
        pltpu.sync_copy(zer_hbm, acc.at[pl.ds(s * MB, MB)])

    @pl.when(s == 10)
    def _():
        pltpu.sync_copy(zer_hbm.at[pl.ds(0, 8)], acc.at[pl.ds(N, 8)])

    plsc.subcore_barrier()

    def edge_loop(table):
        for p in range(NPH):
            pltpu.sync_copy(gidx_hbm.at[s * NPH + p], gidx_v)
            pltpu.sync_copy(dst_hbm.at[s * NPH + p], dst_v)

            @pl.loop(0, CHP)
            def _(j):
                pltpu.async_copy(table.at[gidx_v.at[j]], rows_a, sem_a).wait()
                pltpu.sync_copy(rows_a, acc.at[dst_v.at[j]], add=True)

    @pl.when(c == 0)
    def _():
        edge_loop(hlo_hbm)

    @pl.when(c == 1)
    def _():
        edge_loop(hhi_hbm)

    plsc.subcore_barrier()

    @pl.when(jnp.logical_and(s < 10, c == 0))
    def _():
        pltpu.sync_copy(acc.at[pl.ds(s * MB, MB)], outlo_hbm.at[pl.ds(s * MB, MB)])

    @pl.when(jnp.logical_and(s < 10, c == 1))
    def _():
        pltpu.sync_copy(acc.at[pl.ds(s * MB, MB)], outhi_hbm.at[pl.ds(s * MB, MB)])


@functools.cache
def _sc_agg_kernel():
    mesh = plsc.VectorSubcoreMesh(core_axis_name="c", subcore_axis_name="s",
                                  num_cores=2, num_subcores=NSUB)
    return pl.kernel(
        _sc_agg_body,
        out_type=(
            jax.ShapeDtypeStruct((N, H), jnp.float32),
            jax.ShapeDtypeStruct((N, H), jnp.float32),
        ),
        mesh=mesh,
        scratch_types=[
            pltpu.VMEM((CHP, CHUNK), jnp.int32),
            pltpu.VMEM((CHP, CHUNK), jnp.int32),
            pltpu.VMEM((CHUNK, H), jnp.float32),
            pltpu.VMEM((CHUNK, H), jnp.float32),
            pltpu.VMEM_SHARED((ACC_ROWS, H), jnp.float32),
            pltpu.SemaphoreType.DMA,
            pltpu.SemaphoreType.DMA,
        ],
    )


def _sc_agg(hlo, hhi, gidx3, dst3, zer):
    return _sc_agg_kernel()(hlo, hhi, gidx3, dst3, zer)


def _layer(h, gidx3, dst3, zer, bases, coef, Wself, bias, g, b, relu):
    w_flat = _basis_w(coef, bases.reshape(B, D * D))
    hlo, hhi = _hrel(h, w_flat.reshape(R, D, D))
    agg_lo, agg_hi = _sc_agg(hlo, hhi, gidx3, dst3, zer)
    s = _mm(h, Wself)
    return _ln(agg_lo, agg_hi, s,
               bias.reshape(1, D), g.reshape(1, D), b.reshape(1, D), relu)


def kernel(x, edge_index, etypes, W_in, b_in, bases1, coef1, Wself1, bias1,
           ln1_g, ln1_b, bases2, coef2, Wself2, bias2, ln2_g, ln2_b):
    pad = EPAD - E
    src_p = jnp.pad(edge_index[0].astype(jnp.int32), (0, pad))
    et_p = jnp.pad(etypes.astype(jnp.int32), (0, pad))
    dst_p = jnp.pad(edge_index[1].astype(jnp.int32), (0, pad),
                    constant_values=TRASH)

    gidx2 = _prep_idx(src_p.reshape(EPAD // CHUNK, CHUNK),
                      et_p.reshape(EPAD // CHUNK, CHUNK))
    gidx3 = gidx2.reshape(NSUB * NPH, CHP, CHUNK)
    dst3 = dst_p.reshape(NSUB * NPH, CHP, CHUNK)
    zer = jnp.zeros((MB, H), jnp.float32)

    h = _mm_bias(x, W_in, b_in.reshape(1, D))
    h = _layer(h, gidx3, dst3, zer, bases1, coef1, Wself1, bias1,
               ln1_g, ln1_b, True)
    h = _layer(h, gidx3, dst3, zer, bases2, coef2, Wself2, bias2,
               ln2_g, ln2_b, False)
    return h


# speedup vs baseline: 1.1353x; 1.1353x over previous
import functools

import jax
import jax.numpy as jnp
from jax import lax
from jax.experimental import pallas as pl
from jax.experimental.pallas import tpu as pltpu
from jax.experimental.pallas import tpu_sc as plsc

N = 10000
E = 160000
D = 256
R = 8
B = 8
H = 128

NSUB = 16
CHUNK = 128
CH = 80
PACK = 16384
EPAD = NSUB * CH * CHUNK
TRASH = N
ACC_ROWS = N + 8
MB = 1000
NMB = N // MB


def _prep_body(src_ref, et_ref, dst_ref, g_ref):
    g_ref[...] = (et_ref[...] * N + src_ref[...]) * PACK + dst_ref[...]


def _prep_idx(src2, et2, dst2):
    return pl.pallas_call(
        _prep_body,
        out_shape=jax.ShapeDtypeStruct(src2.shape, jnp.int32),
    )(src2, et2, dst2)


def _mm_bias_body(x_ref, w_ref, b_ref, o_ref):
    xb = x_ref[...].astype(jnp.bfloat16)
    wb = w_ref[...].astype(jnp.bfloat16)
    o_ref[...] = jnp.dot(xb, wb, preferred_element_type=jnp.float32) + b_ref[...]


def _mm_bias(x, w, b2):
    return pl.pallas_call(
        _mm_bias_body,
        grid=(NMB,),
        in_specs=[
            pl.BlockSpec((MB, D), lambda m: (m, 0)),
            pl.BlockSpec((D, D), lambda m: (0, 0)),
            pl.BlockSpec((1, D), lambda m: (0, 0)),
        ],
        out_specs=pl.BlockSpec((MB, D), lambda m: (m, 0)),
        out_shape=jax.ShapeDtypeStruct((N, D), jnp.float32),
    )(x, w, b2)


def _mm_body(x_ref, w_ref, o_ref):
    xb = x_ref[...].astype(jnp.bfloat16)
    wb = w_ref[...].astype(jnp.bfloat16)
    o_ref[...] = jnp.dot(xb, wb, preferred_element_type=jnp.float32)


def _mm(x, w):
    return pl.pallas_call(
        _mm_body,
        grid=(NMB,),
        in_specs=[
            pl.BlockSpec((MB, D), lambda m: (m, 0)),
            pl.BlockSpec((D, D), lambda m: (0, 0)),
        ],
        out_specs=pl.BlockSpec((MB, D), lambda m: (m, 0)),
        out_shape=jax.ShapeDtypeStruct((N, D), jnp.float32),
    )(x, w)


def _basis_body(c_ref, bs_ref, o_ref):
    cb = c_ref[...].astype(jnp.bfloat16)
    bb = bs_ref[...].astype(jnp.bfloat16)
    o_ref[...] = jnp.dot(cb, bb, preferred_element_type=jnp.float32)


def _basis_w(coef, bases_flat):
    cols = 8192
    return pl.pallas_call(
        _basis_body,
        grid=(bases_flat.shape[1] // cols,),
        in_specs=[
            pl.BlockSpec((R, B), lambda i: (0, 0)),
            pl.BlockSpec((B, cols), lambda i: (0, i)),
        ],
        out_specs=pl.BlockSpec((R, cols), lambda i: (0, i)),
        out_shape=jax.ShapeDtypeStruct((R, bases_flat.shape[1]), jnp.float32),
    )(coef, bases_flat)


def _hrel_body(h_ref, w_ref, lo_ref, hi_ref):
    hb = h_ref[...].astype(jnp.bfloat16)
    wb = w_ref[0].astype(jnp.bfloat16)
    o = jnp.dot(hb, wb, preferred_element_type=jnp.float32)
    lo_ref[...] = o[:, :H]
    hi_ref[...] = o[:, H:]


def _hrel(h, w3):
    return pl.pallas_call(
        _hrel_body,
        grid=(NMB, R),
        in_specs=[
            pl.BlockSpec((MB, D), lambda m, r: (m, 0)),
            pl.BlockSpec((1, D, D), lambda m, r: (r, 0, 0)),
        ],
        out_specs=[
            pl.BlockSpec((MB, H), lambda m, r: (r * NMB + m, 0)),
            pl.BlockSpec((MB, H), lambda m, r: (r * NMB + m, 0)),
        ],
        out_shape=[
            jax.ShapeDtypeStruct((R * N, H), jnp.float32),
            jax.ShapeDtypeStruct((R * N, H), jnp.float32),
        ],
    )(h, w3)


def _ln_body(lo_ref, hi_ref, s_ref, bias_ref, g_ref, b_ref, o_ref, *, relu):
    o = jnp.concatenate([lo_ref[...], hi_ref[...]], axis=1) + s_ref[...] + bias_ref[...]
    mu = jnp.mean(o, axis=1, keepdims=True)
    d = o - mu
    var = jnp.mean(d * d, axis=1, keepdims=True)
    y = d * lax.rsqrt(var + 1e-5) * g_ref[...] + b_ref[...]
    if relu:
        y = jnp.maximum(y, 0.0)
    o_ref[...] = y


def _ln(agg_lo, agg_hi, s, bias2, g2, b2, relu):
    return pl.pallas_call(
        functools.partial(_ln_body, relu=relu),
        grid=(NMB,),
        in_specs=[
            pl.BlockSpec((MB, H), lambda m: (m, 0)),
            pl.BlockSpec((MB, H), lambda m: (m, 0)),
            pl.BlockSpec((MB, D), lambda m: (m, 0)),
            pl.BlockSpec((1, D), lambda m: (0, 0)),
            pl.BlockSpec((1, D), lambda m: (0, 0)),
            pl.BlockSpec((1, D), lambda m: (0, 0)),
        ],
        out_specs=pl.BlockSpec((MB, D), lambda m: (m, 0)),
        out_shape=jax.ShapeDtypeStruct((N, D), jnp.float32),
    )(agg_lo, agg_hi, s, bias2, g2, b2)


def _sc_agg_body(hlo_hbm, hhi_hbm, pidx_hbm, zer_hbm,
                 outlo_hbm, outhi_hbm,
                 pidx_v, g_a, d_a, g_b, d_b, rows_a, rows_b, acc,
                 sem_a, sem_b):
    c = lax.axis_index("c")
    s = lax.axis_index("s")

    @pl.when(s < 10)
    def _():
        pltpu.sync_copy(zer_hbm, acc.at[pl.ds(s * MB, MB)])

    @pl.when(s == 10)
    def _():
        pltpu.sync_copy(zer_hbm.at[pl.ds(0, 8)], acc.at[pl.ds(N, 8)])

    pltpu.sync_copy(pidx_hbm.at[s], pidx_v)
    plsc.subcore_barrier()

    def unpack(j, g_ref, d_ref):
        for k in range(CHUNK // 16):
            sl = pl.ds(k * 16, 16)
            p = pidx_v[j, sl]
            g_ref[sl] = lax.shift_right_logical(p, 14)
            d_ref[sl] = lax.bitwise_and(p, PACK - 1)

    def edge_loop(table):
        unpack(0, g_a, d_a)
        pltpu.async_copy(table.at[g_a], rows_a, sem_a)

        @pl.loop(0, CH, step=2)
        def _(j):
            unpack(j + 1, g_b, d_b)
            pltpu.async_copy(table.at[g_b], rows_b, sem_b)
            pltpu.make_async_copy(table.at[g_a], rows_a, sem_a).wait()
            pltpu.sync_copy(rows_a, acc.at[d_a], add=True)

            @pl.when(j + 2 < CH)
            def _():
                unpack(j + 2, g_a, d_a)
                pltpu.async_copy(table.at[g_a], rows_a, sem_a)

            pltpu.make_async_copy(table.at[g_b], rows_b, sem_b).wait()
            pltpu.sync_copy(rows_b, acc.at[d_b], add=True)

    @pl.when(c == 0)
    def _():
        edge_loop(hlo_hbm)

    @pl.when(c == 1)
    def _():
        edge_loop(hhi_hbm)

    plsc.subcore_barrier()

    @pl.when(jnp.logical_and(s < 10, c == 0))
    def _():
        pltpu.sync_copy(acc.at[pl.ds(s * MB, MB)], outlo_hbm.at[pl.ds(s * MB, MB)])

    @pl.when(jnp.logical_and(s < 10, c == 1))
    def _():
        pltpu.sync_copy(acc.at[pl.ds(s * MB, MB)], outhi_hbm.at[pl.ds(s * MB, MB)])


@functools.cache
def _sc_agg_kernel():
    mesh = plsc.VectorSubcoreMesh(core_axis_name="c", subcore_axis_name="s",
                                  num_cores=2, num_subcores=NSUB)
    return pl.kernel(
        _sc_agg_body,
        out_type=(
            jax.ShapeDtypeStruct((N, H), jnp.float32),
            jax.ShapeDtypeStruct((N, H), jnp.float32),
        ),
        mesh=mesh,
        scratch_types=[
            pltpu.VMEM((CH, CHUNK), jnp.int32),
            pltpu.VMEM((CHUNK,), jnp.int32),
            pltpu.VMEM((CHUNK,), jnp.int32),
            pltpu.VMEM((CHUNK,), jnp.int32),
            pltpu.VMEM((CHUNK,), jnp.int32),
            pltpu.VMEM((CHUNK, H), jnp.float32),
            pltpu.VMEM((CHUNK, H), jnp.float32),
            pltpu.VMEM_SHARED((ACC_ROWS, H), jnp.float32),
            pltpu.SemaphoreType.DMA,
            pltpu.SemaphoreType.DMA,
        ],
    )


def _sc_agg(hlo, hhi, pidx3, zer):
    return _sc_agg_kernel()(hlo, hhi, pidx3, zer)


def _layer(h, pidx3, zer, bases, coef, Wself, bias, g, b, relu):
    w_flat = _basis_w(coef, bases.reshape(B, D * D))
    hlo, hhi = _hrel(h, w_flat.reshape(R, D, D))
    agg_lo, agg_hi = _sc_agg(hlo, hhi, pidx3, zer)
    s = _mm(h, Wself)
    return _ln(agg_lo, agg_hi, s,
               bias.reshape(1, D), g.reshape(1, D), b.reshape(1, D), relu)


def kernel(x, edge_index, etypes, W_in, b_in, bases1, coef1, Wself1, bias1,
           ln1_g, ln1_b, bases2, coef2, Wself2, bias2, ln2_g, ln2_b):
    pad = EPAD - E
    src_p = jnp.pad(edge_index[0].astype(jnp.int32), (0, pad))
    et_p = jnp.pad(etypes.astype(jnp.int32), (0, pad))
    dst_p = jnp.pad(edge_index[1].astype(jnp.int32), (0, pad),
                    constant_values=TRASH)

    pidx2 = _prep_idx(src_p.reshape(EPAD // CHUNK, CHUNK),
                      et_p.reshape(EPAD // CHUNK, CHUNK),
                      dst_p.reshape(EPAD // CHUNK, CHUNK))
    pidx3 = pidx2.reshape(NSUB, CH, CHUNK)
    zer = jnp.zeros((MB, H), jnp.float32)

    h = _mm_bias(x, W_in, b_in.reshape(1, D))
    h = _layer(h, pidx3, zer, bases1, coef1, Wself1, bias1,
               ln1_g, ln1_b, True)
    h = _layer(h, pidx3, zer, bases2, coef2, Wself2, bias2,
               ln2_g, ln2_b, False)
    return h


# speedup vs baseline: 1.3177x; 1.1607x over previous
import functools

import jax
import jax.numpy as jnp
from jax import lax
from jax.experimental import pallas as pl
from jax.experimental.pallas import tpu as pltpu
from jax.experimental.pallas import tpu_sc as plsc

N = 10000
E = 160000
D = 256
R = 8
B = 8
H = 128

NSUB = 16
CHUNK = 128
CH = 79
EPAD = NSUB * CH * CHUNK
TRASH = N
ACC_ROWS = N + 8
MB = 1000
NMB = N // MB


def _prep_body(src_ref, et_ref, g_ref):
    g_ref[...] = et_ref[...] * N + src_ref[...]


def _prep_idx(src2, et2):
    return pl.pallas_call(
        _prep_body,
        out_shape=jax.ShapeDtypeStruct(src2.shape, jnp.int32),
    )(src2, et2)


def _mm_bias_body(x_ref, w_ref, b_ref, o_ref):
    xb = x_ref[...].astype(jnp.bfloat16)
    wb = w_ref[...].astype(jnp.bfloat16)
    o_ref[...] = jnp.dot(xb, wb, preferred_element_type=jnp.float32) + b_ref[...]


def _mm_bias(x, w, b2):
    return pl.pallas_call(
        _mm_bias_body,
        grid=(NMB,),
        in_specs=[
            pl.BlockSpec((MB, D), lambda m: (m, 0)),
            pl.BlockSpec((D, D), lambda m: (0, 0)),
            pl.BlockSpec((1, D), lambda m: (0, 0)),
        ],
        out_specs=pl.BlockSpec((MB, D), lambda m: (m, 0)),
        out_shape=jax.ShapeDtypeStruct((N, D), jnp.float32),
    )(x, w, b2)


def _basis_body(c_ref, bs_ref, o_ref):
    cb = c_ref[...].astype(jnp.bfloat16)
    bb = bs_ref[...].astype(jnp.bfloat16)
    o_ref[...] = jnp.dot(cb, bb, preferred_element_type=jnp.float32)


def _basis_w(coef, bases_flat):
    cols = 8192
    return pl.pallas_call(
        _basis_body,
        grid=(bases_flat.shape[1] // cols,),
        in_specs=[
            pl.BlockSpec((R, B), lambda i: (0, 0)),
            pl.BlockSpec((B, cols), lambda i: (0, i)),
        ],
        out_specs=pl.BlockSpec((R, cols), lambda i: (0, i)),
        out_shape=jax.ShapeDtypeStruct((R, bases_flat.shape[1]), jnp.float32),
    )(coef, bases_flat)


def _hrel_body(h_ref, w_ref, lo_ref, hi_ref):
    hb = h_ref[...].astype(jnp.bfloat16)
    wb = w_ref[0].astype(jnp.bfloat16)
    o = jnp.dot(hb, wb, preferred_element_type=jnp.float32)
    lo_ref[...] = o[:, :H]
    hi_ref[...] = o[:, H:]


def _hrel(h, w3):
    return pl.pallas_call(
        _hrel_body,
        grid=(NMB, R),
        in_specs=[
            pl.BlockSpec((MB, D), lambda m, r: (m, 0)),
            pl.BlockSpec((1, D, D), lambda m, r: (r, 0, 0)),
        ],
        out_specs=[
            pl.BlockSpec((MB, H), lambda m, r: (r * NMB + m, 0)),
            pl.BlockSpec((MB, H), lambda m, r: (r * NMB + m, 0)),
        ],
        out_shape=[
            jax.ShapeDtypeStruct((R * N, H), jnp.float32),
            jax.ShapeDtypeStruct((R * N, H), jnp.float32),
        ],
    )(h, w3)


def _ln_body(lo_ref, hi_ref, h_ref, ws_ref, bias_ref, g_ref, b_ref, o_ref, *,
             relu):
    hb = h_ref[...].astype(jnp.bfloat16)
    wb = ws_ref[...].astype(jnp.bfloat16)
    s = jnp.dot(hb, wb, preferred_element_type=jnp.float32)
    o = jnp.concatenate([lo_ref[...], hi_ref[...]], axis=1) + s + bias_ref[...]
    mu = jnp.mean(o, axis=1, keepdims=True)
    d = o - mu
    var = jnp.mean(d * d, axis=1, keepdims=True)
    y = d * lax.rsqrt(var + 1e-5) * g_ref[...] + b_ref[...]
    if relu:
        y = jnp.maximum(y, 0.0)
    o_ref[...] = y


def _ln(agg_lo, agg_hi, h, Wself, bias2, g2, b2, relu):
    return pl.pallas_call(
        functools.partial(_ln_body, relu=relu),
        grid=(NMB,),
        in_specs=[
            pl.BlockSpec((MB, H), lambda m: (m, 0)),
            pl.BlockSpec((MB, H), lambda m: (m, 0)),
            pl.BlockSpec((MB, D), lambda m: (m, 0)),
            pl.BlockSpec((D, D), lambda m: (0, 0)),
            pl.BlockSpec((1, D), lambda m: (0, 0)),
            pl.BlockSpec((1, D), lambda m: (0, 0)),
            pl.BlockSpec((1, D), lambda m: (0, 0)),
        ],
        out_specs=pl.BlockSpec((MB, D), lambda m: (m, 0)),
        out_shape=jax.ShapeDtypeStruct((N, D), jnp.float32),
    )(agg_lo, agg_hi, h, Wself, bias2, g2, b2)


def _sc_agg_body(hlo_hbm, hhi_hbm, gidx_hbm, dst_hbm, zer_hbm,
                 outlo_hbm, outhi_hbm,
                 gidx_v, dst_v, rows_v, acc, sem):
    c = lax.axis_index("c")
    s = lax.axis_index("s")

    @pl.when(s < 10)
    def _():
        pltpu.sync_copy(zer_hbm, acc.at[pl.ds(s * MB, MB)])

    @pl.when(s == 10)
    def _():
        pltpu.sync_copy(zer_hbm.at[pl.ds(0, 8)], acc.at[pl.ds(N, 8)])

    pltpu.sync_copy(gidx_hbm.at[s], gidx_v)
    pltpu.sync_copy(dst_hbm.at[s], dst_v)
    plsc.subcore_barrier()

    def edge_loop(table):
        @pl.loop(0, CH)
        def _(j):
            pltpu.async_copy(table.at[gidx_v.at[j]], rows_v, sem).wait()
            pltpu.sync_copy(rows_v, acc.at[dst_v.at[j]], add=True)

    @pl.when(c == 0)
    def _():
        edge_loop(hlo_hbm)

    @pl.when(c == 1)
    def _():
        edge_loop(hhi_hbm)

    plsc.subcore_barrier()

    @pl.when(jnp.logical_and(s < 10, c == 0))
    def _():
        pltpu.sync_copy(acc.at[pl.ds(s * MB, MB)], outlo_hbm.at[pl.ds(s * MB, MB)])

    @pl.when(jnp.logical_and(s < 10, c == 1))
    def _():
        pltpu.sync_copy(acc.at[pl.ds(s * MB, MB)], outhi_hbm.at[pl.ds(s * MB, MB)])


@functools.cache
def _sc_agg_kernel():
    mesh = plsc.VectorSubcoreMesh(core_axis_name="c", subcore_axis_name="s",
                                  num_cores=2, num_subcores=NSUB)
    return pl.kernel(
        _sc_agg_body,
        out_type=(
            jax.ShapeDtypeStruct((N, H), jnp.float32),
            jax.ShapeDtypeStruct((N, H), jnp.float32),
        ),
        mesh=mesh,
        scratch_types=[
            pltpu.VMEM((CH, CHUNK), jnp.int32),
            pltpu.VMEM((CH, CHUNK), jnp.int32),
            pltpu.VMEM((CHUNK, H), jnp.float32),
            pltpu.VMEM_SHARED((ACC_ROWS, H), jnp.float32),
            pltpu.SemaphoreType.DMA,
        ],
    )


def _sc_agg(hlo, hhi, gidx3, dst3, zer):
    return _sc_agg_kernel()(hlo, hhi, gidx3, dst3, zer)


def _layer(h, gidx3, dst3, zer, bases, coef, Wself, bias, g, b, relu):
    w_flat = _basis_w(coef, bases.reshape(B, D * D))
    hlo, hhi = _hrel(h, w_flat.reshape(R, D, D))
    agg_lo, agg_hi = _sc_agg(hlo, hhi, gidx3, dst3, zer)
    return _ln(agg_lo, agg_hi, h, Wself,
               bias.reshape(1, D), g.reshape(1, D), b.reshape(1, D), relu)


def kernel(x, edge_index, etypes, W_in, b_in, bases1, coef1, Wself1, bias1,
           ln1_g, ln1_b, bases2, coef2, Wself2, bias2, ln2_g, ln2_b):
    pad = EPAD - E
    src_p = jnp.pad(edge_index[0].astype(jnp.int32), (0, pad))
    et_p = jnp.pad(etypes.astype(jnp.int32), (0, pad))
    dst_p = jnp.pad(edge_index[1].astype(jnp.int32), (0, pad),
                    constant_values=TRASH)

    gidx2 = _prep_idx(src_p.reshape(EPAD // CHUNK, CHUNK),
                      et_p.reshape(EPAD // CHUNK, CHUNK))
    gidx3 = gidx2.reshape(NSUB, CH, CHUNK)
    dst3 = dst_p.reshape(NSUB, CH, CHUNK)
    zer = jnp.zeros((MB, H), jnp.float32)

    h = _mm_bias(x, W_in, b_in.reshape(1, D))
    h = _layer(h, gidx3, dst3, zer, bases1, coef1, Wself1, bias1,
               ln1_g, ln1_b, True)
    h = _layer(h, gidx3, dst3, zer, bases2, coef2, Wself2, bias2,
               ln2_g, ln2_b, False)
    return h


# speedup vs baseline: 1.3631x; 1.0344x over previous
import functools

import jax
import jax.numpy as jnp
from jax import lax
from jax.experimental import pallas as pl
from jax.experimental.pallas import tpu as pltpu
from jax.experimental.pallas import tpu_sc as plsc

N = 10000
E = 160000
D = 256
R = 8
B = 8
H = 128

NSUB = 16
CHUNK = 128
CH = 79
EPAD = NSUB * CH * CHUNK
TRASH = N
ACC_ROWS = N + 8
MB = 1000
NMB = N // MB


IB = EPAD // CHUNK


def _basis_w_block(coef_ref, bases_ref, r):
    w = coef_ref[r, 0] * bases_ref[0]
    for b in range(1, B):
        w = w + coef_ref[r, b] * bases_ref[b]
    return w


def _front1_body(x_ref, win_ref, bin_ref, coef_ref, bases_ref,
                 src_ref, et_ref,
                 lo_ref, hi_ref, h_ref, gidx_ref, w_s):
    m = pl.program_id(0)
    r = pl.program_id(1)

    @pl.when(m == 0)
    def _():
        w_s[r] = _basis_w_block(coef_ref, bases_ref, r)

    @pl.when(r == 0)
    def _():
        xb = x_ref[...].astype(jnp.bfloat16)
        wb = win_ref[...].astype(jnp.bfloat16)
        h_ref[...] = (jnp.dot(xb, wb, preferred_element_type=jnp.float32)
                      + bin_ref[...])

    @pl.when(jnp.logical_and(m == 0, r == 0))
    def _():
        gidx_ref[...] = et_ref[...] * N + src_ref[...]

    hb = h_ref[...].astype(jnp.bfloat16)
    wb = w_s[r].astype(jnp.bfloat16)
    o = jnp.dot(hb, wb, preferred_element_type=jnp.float32)
    lo_ref[...] = o[:, :H]
    hi_ref[...] = o[:, H:]


def _front1(x, W_in, b_in2, coef, bases, src2, et2):
    return pl.pallas_call(
        _front1_body,
        grid=(NMB, R),
        in_specs=[
            pl.BlockSpec((MB, D), lambda m, r: (m, 0)),
            pl.BlockSpec((D, D), lambda m, r: (0, 0)),
            pl.BlockSpec((1, D), lambda m, r: (0, 0)),
            pl.BlockSpec(memory_space=pltpu.SMEM),
            pl.BlockSpec((B, D, D), lambda m, r: (0, 0, 0)),
            pl.BlockSpec((IB, CHUNK), lambda m, r: (0, 0)),
            pl.BlockSpec((IB, CHUNK), lambda m, r: (0, 0)),
        ],
        out_specs=[
            pl.BlockSpec((MB, H), lambda m, r: (r * NMB + m, 0)),
            pl.BlockSpec((MB, H), lambda m, r: (r * NMB + m, 0)),
            pl.BlockSpec((MB, D), lambda m, r: (m, 0)),
            pl.BlockSpec((IB, CHUNK), lambda m, r: (0, 0)),
        ],
        out_shape=[
            jax.ShapeDtypeStruct((R * N, H), jnp.float32),
            jax.ShapeDtypeStruct((R * N, H), jnp.float32),
            jax.ShapeDtypeStruct((N, D), jnp.float32),
            jax.ShapeDtypeStruct((IB, CHUNK), jnp.int32),
        ],
        scratch_shapes=[pltpu.VMEM((R, D, D), jnp.float32)],
    )(x, W_in, b_in2, coef, bases, src2, et2)


def _front2_body(h_ref, coef_ref, bases_ref, lo_ref, hi_ref, w_s):
    m = pl.program_id(0)
    r = pl.program_id(1)

    @pl.when(m == 0)
    def _():
        w_s[r] = _basis_w_block(coef_ref, bases_ref, r)

    hb = h_ref[...].astype(jnp.bfloat16)
    wb = w_s[r].astype(jnp.bfloat16)
    o = jnp.dot(hb, wb, preferred_element_type=jnp.float32)
    lo_ref[...] = o[:, :H]
    hi_ref[...] = o[:, H:]


def _front2(h, coef, bases):
    return pl.pallas_call(
        _front2_body,
        grid=(NMB, R),
        in_specs=[
            pl.BlockSpec((MB, D), lambda m, r: (m, 0)),
            pl.BlockSpec(memory_space=pltpu.SMEM),
            pl.BlockSpec((B, D, D), lambda m, r: (0, 0, 0)),
        ],
        out_specs=[
            pl.BlockSpec((MB, H), lambda m, r: (r * NMB + m, 0)),
            pl.BlockSpec((MB, H), lambda m, r: (r * NMB + m, 0)),
        ],
        out_shape=[
            jax.ShapeDtypeStruct((R * N, H), jnp.float32),
            jax.ShapeDtypeStruct((R * N, H), jnp.float32),
        ],
        scratch_shapes=[pltpu.VMEM((R, D, D), jnp.float32)],
    )(h, coef, bases)


def _ln_body(lo_ref, hi_ref, h_ref, ws_ref, bias_ref, g_ref, b_ref, o_ref, *,
             relu):
    hb = h_ref[...].astype(jnp.bfloat16)
    wb = ws_ref[...].astype(jnp.bfloat16)
    s = jnp.dot(hb, wb, preferred_element_type=jnp.float32)
    o = jnp.concatenate([lo_ref[...], hi_ref[...]], axis=1) + s + bias_ref[...]
    mu = jnp.mean(o, axis=1, keepdims=True)
    d = o - mu
    var = jnp.mean(d * d, axis=1, keepdims=True)
    y = d * lax.rsqrt(var + 1e-5) * g_ref[...] + b_ref[...]
    if relu:
        y = jnp.maximum(y, 0.0)
    o_ref[...] = y


def _ln(agg_lo, agg_hi, h, Wself, bias2, g2, b2, relu):
    return pl.pallas_call(
        functools.partial(_ln_body, relu=relu),
        grid=(NMB,),
        in_specs=[
            pl.BlockSpec((MB, H), lambda m: (m, 0)),
            pl.BlockSpec((MB, H), lambda m: (m, 0)),
            pl.BlockSpec((MB, D), lambda m: (m, 0)),
            pl.BlockSpec((D, D), lambda m: (0, 0)),
            pl.BlockSpec((1, D), lambda m: (0, 0)),
            pl.BlockSpec((1, D), lambda m: (0, 0)),
            pl.BlockSpec((1, D), lambda m: (0, 0)),
        ],
        out_specs=pl.BlockSpec((MB, D), lambda m: (m, 0)),
        out_shape=jax.ShapeDtypeStruct((N, D), jnp.float32),
    )(agg_lo, agg_hi, h, Wself, bias2, g2, b2)


def _sc_agg_body(hlo_hbm, hhi_hbm, gidx_hbm, dst_hbm, zer_hbm,
                 outlo_hbm, outhi_hbm,
                 gidx_v, dst_v, rows_v, acc, sem):
    c = lax.axis_index("c")
    s = lax.axis_index("s")

    @pl.when(s < 10)
    def _():
        pltpu.sync_copy(zer_hbm, acc.at[pl.ds(s * MB, MB)])

    @pl.when(s == 10)
    def _():
        pltpu.sync_copy(zer_hbm.at[pl.ds(0, 8)], acc.at[pl.ds(N, 8)])

    pltpu.sync_copy(gidx_hbm.at[s], gidx_v)
    pltpu.sync_copy(dst_hbm.at[s], dst_v)
    plsc.subcore_barrier()

    def edge_loop(table):
        @pl.loop(0, CH)
        def _(j):
            pltpu.async_copy(table.at[gidx_v.at[j]], rows_v, sem).wait()
            pltpu.sync_copy(rows_v, acc.at[dst_v.at[j]], add=True)

    @pl.when(c == 0)
    def _():
        edge_loop(hlo_hbm)

    @pl.when(c == 1)
    def _():
        edge_loop(hhi_hbm)

    plsc.subcore_barrier()

    @pl.when(jnp.logical_and(s < 10, c == 0))
    def _():
        pltpu.sync_copy(acc.at[pl.ds(s * MB, MB)], outlo_hbm.at[pl.ds(s * MB, MB)])

    @pl.when(jnp.logical_and(s < 10, c == 1))
    def _():
        pltpu.sync_copy(acc.at[pl.ds(s * MB, MB)], outhi_hbm.at[pl.ds(s * MB, MB)])


@functools.cache
def _sc_agg_kernel():
    mesh = plsc.VectorSubcoreMesh(core_axis_name="c", subcore_axis_name="s",
                                  num_cores=2, num_subcores=NSUB)
    return pl.kernel(
        _sc_agg_body,
        out_type=(
            jax.ShapeDtypeStruct((N, H), jnp.float32),
            jax.ShapeDtypeStruct((N, H), jnp.float32),
        ),
        mesh=mesh,
        scratch_types=[
            pltpu.VMEM((CH, CHUNK), jnp.int32),
            pltpu.VMEM((CH, CHUNK), jnp.int32),
            pltpu.VMEM((CHUNK, H), jnp.float32),
            pltpu.VMEM_SHARED((ACC_ROWS, H), jnp.float32),
            pltpu.SemaphoreType.DMA,
        ],
    )


def _sc_agg(hlo, hhi, gidx3, dst3, zer):
    return _sc_agg_kernel()(hlo, hhi, gidx3, dst3, zer)


def _rnd(a):
    return a.astype(jnp.bfloat16).astype(jnp.float32)


def kernel(x, edge_index, etypes, W_in, b_in, bases1, coef1, Wself1, bias1,
           ln1_g, ln1_b, bases2, coef2, Wself2, bias2, ln2_g, ln2_b):
    pad = EPAD - E
    src_p = jnp.pad(edge_index[0].astype(jnp.int32), (0, pad))
    et_p = jnp.pad(etypes.astype(jnp.int32), (0, pad))
    dst_p = jnp.pad(edge_index[1].astype(jnp.int32), (0, pad),
                    constant_values=TRASH)
    src2 = src_p.reshape(IB, CHUNK)
    et2 = et_p.reshape(IB, CHUNK)
    dst3 = dst_p.reshape(NSUB, CH, CHUNK)
    zer = jnp.zeros((MB, H), jnp.float32)

    hlo, hhi, h, gidx2 = _front1(x, W_in, b_in.reshape(1, D),
                                 _rnd(coef1), _rnd(bases1), src2, et2)
    gidx3 = gidx2.reshape(NSUB, CH, CHUNK)
    agg_lo, agg_hi = _sc_agg(hlo, hhi, gidx3, dst3, zer)
    h = _ln(agg_lo, agg_hi, h, Wself1, bias1.reshape(1, D),
            ln1_g.reshape(1, D), ln1_b.reshape(1, D), True)

    hlo, hhi = _front2(h, _rnd(coef2), _rnd(bases2))
    agg_lo, agg_hi = _sc_agg(hlo, hhi, gidx3, dst3, zer)
    h = _ln(agg_lo, agg_hi, h, Wself2, bias2.reshape(1, D),
            ln2_g.reshape(1, D), ln2_b.reshape(1, D), False)
    return h


# speedup vs baseline: 1.3921x; 1.0213x over previous
import functools

import jax
import jax.numpy as jnp
from jax import lax
from jax.experimental import pallas as pl
from jax.experimental.pallas import tpu as pltpu
from jax.experimental.pallas import tpu_sc as plsc

N = 10000
E = 160000
D = 256
R = 8
B = 8
H = 128

NSUB = 16
CHUNK = 128
CH = 79
EPAD = NSUB * CH * CHUNK
TRASH = N
ACC_ROWS = N + 8
MB = 1000
NMB = N // MB


IB = EPAD // CHUNK


def _basis_w_block(coef_ref, bases_ref, r):
    w = coef_ref[r, 0] * bases_ref[0]
    for b in range(1, B):
        w = w + coef_ref[r, b] * bases_ref[b]
    return w


def _front1_body(x_ref, win_ref, bin_ref, coef_ref, bases_ref,
                 src_ref, et_ref,
                 lo_ref, hi_ref, h_ref, gidx_ref, w_s):
    m = pl.program_id(0)
    r = pl.program_id(1)

    @pl.when(m == 0)
    def _():
        w_s[r] = _basis_w_block(coef_ref, bases_ref, r)

    @pl.when(r == 0)
    def _():
        xb = x_ref[...].astype(jnp.bfloat16)
        wb = win_ref[...].astype(jnp.bfloat16)
        h_ref[...] = (jnp.dot(xb, wb, preferred_element_type=jnp.float32)
                      + bin_ref[...])

    @pl.when(jnp.logical_and(m == 0, r == 0))
    def _():
        gidx_ref[...] = et_ref[...] * N + src_ref[...]

    hb = h_ref[...].astype(jnp.bfloat16)
    wb = w_s[r].astype(jnp.bfloat16)
    o = jnp.dot(hb, wb, preferred_element_type=jnp.float32)
    lo_ref[...] = o[:, :H]
    hi_ref[...] = o[:, H:]


def _front1(x, W_in, b_in2, coef, bases, src2, et2):
    return pl.pallas_call(
        _front1_body,
        grid=(NMB, R),
        in_specs=[
            pl.BlockSpec((MB, D), lambda m, r: (m, 0)),
            pl.BlockSpec((D, D), lambda m, r: (0, 0)),
            pl.BlockSpec((1, D), lambda m, r: (0, 0)),
            pl.BlockSpec(memory_space=pltpu.SMEM),
            pl.BlockSpec((B, D, D), lambda m, r: (0, 0, 0)),
            pl.BlockSpec((IB, CHUNK), lambda m, r: (0, 0)),
            pl.BlockSpec((IB, CHUNK), lambda m, r: (0, 0)),
        ],
        out_specs=[
            pl.BlockSpec((MB, H), lambda m, r: (r * NMB + m, 0)),
            pl.BlockSpec((MB, H), lambda m, r: (r * NMB + m, 0)),
            pl.BlockSpec((MB, D), lambda m, r: (m, 0)),
            pl.BlockSpec((IB, CHUNK), lambda m, r: (0, 0)),
        ],
        out_shape=[
            jax.ShapeDtypeStruct((R * N, H), jnp.float32),
            jax.ShapeDtypeStruct((R * N, H), jnp.float32),
            jax.ShapeDtypeStruct((N, D), jnp.float32),
            jax.ShapeDtypeStruct((IB, CHUNK), jnp.int32),
        ],
        scratch_shapes=[pltpu.VMEM((R, D, D), jnp.float32)],
    )(x, W_in, b_in2, coef, bases, src2, et2)


def _front2_body(h_ref, coef_ref, bases_ref, lo_ref, hi_ref, w_s):
    m = pl.program_id(0)
    r = pl.program_id(1)

    @pl.when(m == 0)
    def _():
        w_s[r] = _basis_w_block(coef_ref, bases_ref, r)

    hb = h_ref[...].astype(jnp.bfloat16)
    wb = w_s[r].astype(jnp.bfloat16)
    o = jnp.dot(hb, wb, preferred_element_type=jnp.float32)
    lo_ref[...] = o[:, :H]
    hi_ref[...] = o[:, H:]


def _front2(h, coef, bases):
    return pl.pallas_call(
        _front2_body,
        grid=(NMB, R),
        in_specs=[
            pl.BlockSpec((MB, D), lambda m, r: (m, 0)),
            pl.BlockSpec(memory_space=pltpu.SMEM),
            pl.BlockSpec((B, D, D), lambda m, r: (0, 0, 0)),
        ],
        out_specs=[
            pl.BlockSpec((MB, H), lambda m, r: (r * NMB + m, 0)),
            pl.BlockSpec((MB, H), lambda m, r: (r * NMB + m, 0)),
        ],
        out_shape=[
            jax.ShapeDtypeStruct((R * N, H), jnp.float32),
            jax.ShapeDtypeStruct((R * N, H), jnp.float32),
        ],
        scratch_shapes=[pltpu.VMEM((R, D, D), jnp.float32)],
    )(h, coef, bases)


def _ln_body(lo_ref, hi_ref, h_ref, ws_ref, bias_ref, g_ref, b_ref, o_ref, *,
             relu):
    hb = h_ref[...].astype(jnp.bfloat16)
    wb = ws_ref[...].astype(jnp.bfloat16)
    s = jnp.dot(hb, wb, preferred_element_type=jnp.float32)
    o = jnp.concatenate([lo_ref[...], hi_ref[...]], axis=1) + s + bias_ref[...]
    mu = jnp.mean(o, axis=1, keepdims=True)
    d = o - mu
    var = jnp.mean(d * d, axis=1, keepdims=True)
    y = d * lax.rsqrt(var + 1e-5) * g_ref[...] + b_ref[...]
    if relu:
        y = jnp.maximum(y, 0.0)
    o_ref[...] = y


def _ln(agg_lo, agg_hi, h, Wself, bias2, g2, b2, relu):
    return pl.pallas_call(
        functools.partial(_ln_body, relu=relu),
        grid=(NMB,),
        in_specs=[
            pl.BlockSpec((MB, H), lambda m: (m, 0)),
            pl.BlockSpec((MB, H), lambda m: (m, 0)),
            pl.BlockSpec((MB, D), lambda m: (m, 0)),
            pl.BlockSpec((D, D), lambda m: (0, 0)),
            pl.BlockSpec((1, D), lambda m: (0, 0)),
            pl.BlockSpec((1, D), lambda m: (0, 0)),
            pl.BlockSpec((1, D), lambda m: (0, 0)),
        ],
        out_specs=pl.BlockSpec((MB, D), lambda m: (m, 0)),
        out_shape=jax.ShapeDtypeStruct((N, D), jnp.float32),
    )(agg_lo, agg_hi, h, Wself, bias2, g2, b2)


def _sc_agg_body(hlo_hbm, hhi_hbm, gidx_hbm, dst_hbm, zer_hbm,
                 outlo_hbm, outhi_hbm,
                 gidx_v, dst_v, rows_v, acc, sem):
    c = lax.axis_index("c")
    s = lax.axis_index("s")

    @pl.when(s < 10)
    def _():
        pltpu.sync_copy(zer_hbm, acc.at[pl.ds(s * MB, MB)])

    @pl.when(s == 10)
    def _():
        pltpu.sync_copy(zer_hbm.at[pl.ds(0, 8)], acc.at[pl.ds(N, 8)])

    pltpu.sync_copy(gidx_hbm.at[s], gidx_v)
    pltpu.sync_copy(dst_hbm.at[s], dst_v)
    plsc.subcore_barrier()

    def edge_loop(table):
        @pl.loop(0, CH)
        def _(j):
            pltpu.async_copy(table.at[gidx_v.at[j]], rows_v, sem).wait()
            pltpu.sync_copy(rows_v, acc.at[dst_v.at[j]], add=True)

    @pl.when(c == 0)
    def _():
        edge_loop(hlo_hbm)

    @pl.when(c == 1)
    def _():
        edge_loop(hhi_hbm)

    plsc.subcore_barrier()

    @pl.when(jnp.logical_and(s < 10, c == 0))
    def _():
        pltpu.sync_copy(acc.at[pl.ds(s * MB, MB)], outlo_hbm.at[pl.ds(s * MB, MB)])

    @pl.when(jnp.logical_and(s < 10, c == 1))
    def _():
        pltpu.sync_copy(acc.at[pl.ds(s * MB, MB)], outhi_hbm.at[pl.ds(s * MB, MB)])


@functools.cache
def _sc_agg_kernel():
    mesh = plsc.VectorSubcoreMesh(core_axis_name="c", subcore_axis_name="s",
                                  num_cores=2, num_subcores=NSUB)
    return pl.kernel(
        _sc_agg_body,
        out_type=(
            jax.ShapeDtypeStruct((N, H), jnp.float32),
            jax.ShapeDtypeStruct((N, H), jnp.float32),
        ),
        mesh=mesh,
        scratch_types=[
            pltpu.VMEM((CH, CHUNK), jnp.int32),
            pltpu.VMEM((CH, CHUNK), jnp.int32),
            pltpu.VMEM((CHUNK, H), jnp.float32),
            pltpu.VMEM_SHARED((ACC_ROWS, H), jnp.float32),
            pltpu.SemaphoreType.DMA,
        ],
    )


def _sc_agg(hlo, hhi, gidx3, dst3, zer):
    return _sc_agg_kernel()(hlo, hhi, gidx3, dst3, zer)


def kernel(x, edge_index, etypes, W_in, b_in, bases1, coef1, Wself1, bias1,
           ln1_g, ln1_b, bases2, coef2, Wself2, bias2, ln2_g, ln2_b):
    pad = EPAD - E
    src_p = jnp.pad(edge_index[0].astype(jnp.int32), (0, pad))
    et_p = jnp.pad(etypes.astype(jnp.int32), (0, pad))
    dst_p = jnp.pad(edge_index[1].astype(jnp.int32), (0, pad),
                    constant_values=TRASH)
    src2 = src_p.reshape(IB, CHUNK)
    et2 = et_p.reshape(IB, CHUNK)
    dst3 = dst_p.reshape(NSUB, CH, CHUNK)
    zer = jnp.zeros((MB, H), jnp.float32)

    hlo, hhi, h, gidx2 = _front1(x, W_in, b_in.reshape(1, D),
                                 coef1, bases1, src2, et2)
    gidx3 = gidx2.reshape(NSUB, CH, CHUNK)
    agg_lo, agg_hi = _sc_agg(hlo, hhi, gidx3, dst3, zer)
    h = _ln(agg_lo, agg_hi, h, Wself1, bias1.reshape(1, D),
            ln1_g.reshape(1, D), ln1_b.reshape(1, D), True)

    hlo, hhi = _front2(h, coef2, bases2)
    agg_lo, agg_hi = _sc_agg(hlo, hhi, gidx3, dst3, zer)
    h = _ln(agg_lo, agg_hi, h, Wself2, bias2.reshape(1, D),
            ln2_g.reshape(1, D), ln2_b.reshape(1, D), False)
    return h
